# Initial kernel scaffold; baseline (speedup 1.0000x reference)
import jax
import jax.numpy as jnp
from jax.experimental import pallas as pl


def kernel(hidden_states, router_logits, w13_weight, w2_weight, w13_bias, w2_bias):
    raise NotImplementedError("write your pallas kernel here")


# speedup vs baseline: 1.1104x; 1.1104x over previous
import functools

import jax
import jax.numpy as jnp
from jax.experimental import pallas as pl
from jax.experimental.pallas import tpu as pltpu

TOPK = 2
BLOCK = 128


def _moe_block_kernel(
    blk_e_ref,
    tok_ref,
    nb_ref,
    hidden_ref,
    w13_ref,
    w2_ref,
    b13_ref,
    b2_ref,
    wslot_ref,
    out_ref,
    x_ref,
    s_ref,
):
    b = pl.program_id(0)
    I = w2_ref.shape[2]

    @pl.when(b == 0)
    def _init():
        out_ref[...] = jnp.zeros_like(out_ref)

    @pl.when(b < nb_ref[0])
    def _body():
        base = b * BLOCK

        def gather(i, _):
            t = tok_ref[base + i]
            x_ref[pl.ds(i, 1), :] = hidden_ref[pl.ds(t, 1), :]
            return 0

        jax.lax.fori_loop(0, BLOCK, gather, 0, unroll=8)

        x = x_ref[...]
        h13 = jax.lax.dot_general(
            x, w13_ref[0],
            (((1,), (1,)), ((), ())),
            preferred_element_type=jnp.float32,
        ) + b13_ref[0]
        gate = h13[:, :I]
        up = h13[:, I:]
        act = gate * jax.lax.logistic(gate) * up
        out_b = jax.lax.dot_general(
            act, w2_ref[0],
            (((1,), (1,)), ((), ())),
            preferred_element_type=jnp.float32,
        ) + b2_ref[0]
        s_ref[...] = out_b * wslot_ref[0, 0, :][:, None]

        def scatter(i, _):
            t = tok_ref[base + i]
            out_ref[pl.ds(t, 1), :] += s_ref[pl.ds(i, 1), :]
            return 0

        jax.lax.fori_loop(0, BLOCK, scatter, 0, unroll=8)


def kernel(hidden_states, router_logits, w13_weight, w2_weight, w13_bias, w2_bias):
    T, H = hidden_states.shape
    E, I2, _ = w13_weight.shape
    nslot_raw = T * TOPK
    NSLOT = nslot_raw + E * BLOCK
    NSLOT = ((NSLOT + BLOCK - 1) // BLOCK) * BLOCK
    NBLK = NSLOT // BLOCK

    probs = jax.nn.softmax(router_logits.astype(jnp.float32), axis=-1)
    topw, topi = jax.lax.top_k(probs, TOPK)
    topw = topw / jnp.sum(topw, axis=-1, keepdims=True)

    flat_e = topi.reshape(-1)
    flat_w = topw.reshape(-1).astype(hidden_states.dtype)
    flat_t = jnp.arange(nslot_raw, dtype=jnp.int32) // TOPK
    onehot = (flat_e[:, None] == jnp.arange(E, dtype=flat_e.dtype)[None, :])
    counts = jnp.sum(onehot.astype(jnp.int32), axis=0)
    rank = jnp.sum(jnp.cumsum(onehot.astype(jnp.int32), axis=0) * onehot,
                   axis=1) - 1
    nblk_e = (counts + BLOCK - 1) // BLOCK
    ends = jnp.cumsum(nblk_e)
    blk_start = ends - nblk_e
    pos = blk_start[flat_e] * BLOCK + rank
    tok_sorted = jnp.zeros((NSLOT,), jnp.int32).at[pos].set(flat_t)
    w_sorted = jnp.zeros((NSLOT,), hidden_states.dtype).at[pos].set(flat_w)
    j = jnp.arange(NBLK, dtype=jnp.int32)
    blk_e = jnp.minimum(jnp.sum((j[:, None] >= ends[None, :]).astype(jnp.int32),
                                axis=1), E - 1).astype(jnp.int32)
    n_active = ends[-1:].astype(jnp.int32)

    grid_spec = pltpu.PrefetchScalarGridSpec(
        num_scalar_prefetch=3,
        grid=(NBLK,),
        in_specs=[
            pl.BlockSpec((T, H), lambda b, be, tok, nb: (0, 0)),
            pl.BlockSpec((1, I2, H), lambda b, be, tok, nb: (be[b], 0, 0)),
            pl.BlockSpec((1, H, I2 // 2), lambda b, be, tok, nb: (be[b], 0, 0)),
            pl.BlockSpec((1, 1, I2), lambda b, be, tok, nb: (be[b], 0, 0)),
            pl.BlockSpec((1, 1, H), lambda b, be, tok, nb: (be[b], 0, 0)),
            pl.BlockSpec((1, 1, BLOCK), lambda b, be, tok, nb: (b, 0, 0)),
        ],
        out_specs=pl.BlockSpec((T, H), lambda b, be, tok, nb: (0, 0)),
        scratch_shapes=[pltpu.VMEM((BLOCK, H), jnp.float32),
                        pltpu.VMEM((BLOCK, H), jnp.float32)],
    )

    out = pl.pallas_call(
        _moe_block_kernel,
        grid_spec=grid_spec,
        out_shape=jax.ShapeDtypeStruct((T, H), hidden_states.dtype),
        compiler_params=pltpu.CompilerParams(
            dimension_semantics=("arbitrary",),
        ),
    )(
        blk_e, tok_sorted, n_active,
        hidden_states, w13_weight, w2_weight,
        w13_bias.reshape(E, 1, I2), w2_bias.reshape(E, 1, H),
        w_sorted.reshape(NBLK, 1, BLOCK),
    )
    return out
